# Initial kernel scaffold; baseline (speedup 1.0000x reference)
import jax
import jax.numpy as jnp
from jax.experimental import pallas as pl


def kernel(features, adj, W1, a_src1, a_dst1, W2, a_src2, a_dst2, W3, a_src3, a_dst3):
    raise NotImplementedError("write your pallas kernel here")


# speedup vs baseline: 1.7762x; 1.7762x over previous
import functools

import jax
import jax.numpy as jnp
from jax.experimental import pallas as pl

_BM = 256


def _proj_body(x_ref, w_ref, wh_ref):
    wh_ref[0] = jnp.dot(x_ref[...], w_ref[0], preferred_element_type=jnp.float32)


def _attn_body(adj_ref, wh_ref, asrc_ref, adst_ref, out_ref, *, bm, heads, hid):
    i = pl.program_id(0)
    adjb = adj_ref[...] > 0.0
    for h in range(heads):
        wh = wh_ref[h]
        asrc = asrc_ref[h][:, None]
        adst = adst_ref[h][:, None]
        rows = wh_ref[h, pl.ds(i * bm, bm), :]
        f_src = jnp.dot(rows, asrc, preferred_element_type=jnp.float32)
        f_dst = jnp.dot(wh, adst, preferred_element_type=jnp.float32)
        s = f_src + f_dst.reshape(1, -1)
        e = jnp.maximum(s, 0.2 * s)
        e = jnp.where(adjb, e, jnp.float32(-9e15))
        m = jnp.max(e, axis=1, keepdims=True)
        p = jnp.exp(e - m)
        denom = jnp.sum(p, axis=1, keepdims=True)
        pv = jnp.dot(p, wh, preferred_element_type=jnp.float32)
        o = pv / denom
        o = jnp.where(o > 0.0, o, jnp.exp(o) - 1.0)
        out_ref[:, h * hid : (h + 1) * hid] = o


def _gat_layer(x, adj, W, a_src, a_dst):
    n = x.shape[0]
    fin = x.shape[1]
    heads, _, hid = W.shape
    bm = min(_BM, n)

    wh = pl.pallas_call(
        _proj_body,
        grid=(heads,),
        in_specs=[
            pl.BlockSpec((n, fin), lambda h: (0, 0)),
            pl.BlockSpec((1, fin, hid), lambda h: (h, 0, 0)),
        ],
        out_specs=pl.BlockSpec((1, n, hid), lambda h: (h, 0, 0)),
        out_shape=jax.ShapeDtypeStruct((heads, n, hid), jnp.float32),
    )(x, W)

    out = pl.pallas_call(
        functools.partial(_attn_body, bm=bm, heads=heads, hid=hid),
        grid=(n // bm,),
        in_specs=[
            pl.BlockSpec((bm, n), lambda i: (i, 0)),
            pl.BlockSpec((heads, n, hid), lambda i: (0, 0, 0)),
            pl.BlockSpec((heads, hid), lambda i: (0, 0)),
            pl.BlockSpec((heads, hid), lambda i: (0, 0)),
        ],
        out_specs=pl.BlockSpec((bm, heads * hid), lambda i: (i, 0)),
        out_shape=jax.ShapeDtypeStruct((n, heads * hid), jnp.float32),
    )(adj, wh, a_src, a_dst)
    return out


@jax.jit
def kernel(features, adj, W1, a_src1, a_dst1, W2, a_src2, a_dst2, W3, a_src3, a_dst3):
    x = _gat_layer(features, adj, W1, a_src1, a_dst1)
    x = _gat_layer(x, adj, W2, a_src2, a_dst2)
    x = _gat_layer(x, adj, W3, a_src3, a_dst3)
    return x
